# baseline (speedup 1.0000x reference)
import functools

import jax
import jax.numpy as jnp
from jax import lax
from jax.experimental import pallas as pl
from jax.experimental.pallas import tpu as pltpu
from jax.experimental.pallas import tpu_sc as plsc

f32 = jnp.float32
i32 = jnp.int32

N = 10000
E = 320000
DIM = 128
H = 8
DK = 16

NC = 2
NS = 16
NW = NC * NS
EW = E // NW
CH = 80
NCH = EW // CH

N_TAB = 10240
NPT = N_TAB // NS
ZR = 128

_mesh = plsc.VectorSubcoreMesh(core_axis_name="c", subcore_axis_name="s")
_sc_params = pltpu.CompilerParams(use_tc_tiling_on_sc=False)


_NB = 2
_NCH0 = (NCH // _NB) * _NB


def _sc_gather2(tab1, tab2, idx1, idx2):
    d1 = tab1.shape[1]
    d2 = tab2.shape[1]

    @functools.partial(
        pl.kernel,
        mesh=_mesh,
        out_type=[
            jax.ShapeDtypeStruct((E, d1), f32),
            jax.ShapeDtypeStruct((E, d2), f32),
        ],
        scratch_types=[
            pltpu.VMEM((_NB, CH), i32),
            pltpu.VMEM((_NB, CH), i32),
            pltpu.VMEM((_NB, CH, d1), f32),
            pltpu.VMEM((_NB, CH, d2), f32),
            pltpu.SemaphoreType.DMA,
            pltpu.SemaphoreType.DMA,
            pltpu.SemaphoreType.DMA,
            pltpu.SemaphoreType.DMA,
        ],
        compiler_params=_sc_params,
    )
    def k(t1, t2, x1, x2, o1, o2, iv1, iv2, b1, b2, sg0, sg1, so0, so1):
        sg = (sg0, sg1)
        so = (so0, so1)
        wid = lax.axis_index("s") * NC + lax.axis_index("c")
        base = wid * EW

        def drain_out(b, off):
            pltpu.make_async_copy(b1.at[b], o1.at[pl.ds(off, CH)], so[b]).wait()
            pltpu.make_async_copy(b2.at[b], o2.at[pl.ds(off, CH)], so[b]).wait()

        def chunk_body(c0, first, width):
            offs = [pl.multiple_of(base + (c0 + b) * CH, 8) for b in range(width)]
            for b in range(width):
                if not first:
                    @pl.when(c0 > 0)
                    def _(b=b, off=offs[b]):
                        drain_out(b, off)
                pltpu.sync_copy(x1.at[pl.ds(offs[b], CH)], iv1.at[b])
                pltpu.sync_copy(x2.at[pl.ds(offs[b], CH)], iv2.at[b])
                pltpu.async_copy(t1.at[iv1.at[b]], b1.at[b], sg[b])
                pltpu.async_copy(t2.at[iv2.at[b]], b2.at[b], sg[b])
            for b in range(width):
                pltpu.make_async_copy(t1.at[iv1.at[b]], b1.at[b], sg[b]).wait()
                pltpu.make_async_copy(t2.at[iv2.at[b]], b2.at[b], sg[b]).wait()
                pltpu.async_copy(b1.at[b], o1.at[pl.ds(offs[b], CH)], so[b])
                pltpu.async_copy(b2.at[b], o2.at[pl.ds(offs[b], CH)], so[b])

        @pl.loop(0, _NCH0, step=_NB)
        def _(c0):
            chunk_body(c0, False, _NB)

        for b in range(_NB):
            off = pl.multiple_of(base + (_NCH0 - _NB + b) * CH, 8)
            drain_out(b, off)

        if _NCH0 < NCH:
            chunk_body(_NCH0, True, NCH - _NCH0)
            for b in range(NCH - _NCH0):
                off = pl.multiple_of(base + (_NCH0 + b) * CH, 8)
                drain_out(b, off)

    return k(tab1, tab2, idx1, idx2)


def _sc_gather_weight_scatter(tab_nodes, j_idx, i_idx, e16):
    NBF = 2
    NCHF = (NCH // NBF) * NBF

    @functools.partial(
        pl.kernel,
        mesh=_mesh,
        out_type=jax.ShapeDtypeStruct((NC, N_TAB, DIM), f32),
        scratch_types=[
            pltpu.VMEM((NBF, CH), i32),
            pltpu.VMEM((NBF, CH), i32),
            pltpu.VMEM((NBF, CH, 16), f32),
            pltpu.VMEM((NBF, CH, DIM), f32),
            pltpu.VMEM((ZR, DIM), f32),
            pltpu.VMEM_SHARED((N_TAB, DIM), f32),
        ] + [pltpu.SemaphoreType.DMA] * (2 * NBF),
        compiler_params=_sc_params,
    )
    def k(tn, jr, ir, er, out, jv, iv, ebuf, buf, z, tab, *sems):
        sl = sems[:NBF]
        sg = sems[NBF:2 * NBF]
        cid = lax.axis_index("c")
        sid = lax.axis_index("s")
        wid = sid * NC + cid
        base = wid * EW

        @pl.loop(0, ZR)
        def _(r):
            for c0 in range(DIM // 16):
                z[r, pl.ds(c0 * 16, 16)] = jnp.zeros((16,), f32)

        @pl.loop(0, NPT // ZR)
        def _(b):
            r0 = pl.multiple_of(sid * NPT + b * ZR, 8)
            pltpu.sync_copy(z, tab.at[pl.ds(r0, ZR)])

        plsc.subcore_barrier()

        def weight(b):
            @pl.loop(0, CH)
            def _(r):
                ev = ebuf[b, r]
                for h in range(H):
                    cols = pl.ds(h * DK, DK)
                    buf[b, r, cols] = buf[b, r, cols] * ev[h]

        def fire_loads(b, off):
            pltpu.async_copy(jr.at[pl.ds(off, CH)], jv.at[b], sl[b])
            pltpu.async_copy(ir.at[pl.ds(off, CH)], iv.at[b], sl[b])
            pltpu.async_copy(er.at[pl.ds(off, CH)], ebuf.at[b], sl[b])

        def wait_loads(b, off):
            pltpu.make_async_copy(jr.at[pl.ds(off, CH)], jv.at[b], sl[b]).wait()
            pltpu.make_async_copy(ir.at[pl.ds(off, CH)], iv.at[b], sl[b]).wait()
            pltpu.make_async_copy(er.at[pl.ds(off, CH)], ebuf.at[b], sl[b]).wait()

        @pl.loop(0, NCHF, step=NBF)
        def _(c0):
            offs = [pl.multiple_of(base + (c0 + b) * CH, 8) for b in range(NBF)]
            for b in range(NBF):
                fire_loads(b, offs[b])
            for b in range(NBF):
                wait_loads(b, offs[b])
                pltpu.async_copy(tn.at[iv.at[b]], buf.at[b], sg[b])
            for b in range(NBF):
                pltpu.make_async_copy(tn.at[iv.at[b]], buf.at[b], sg[b]).wait()
                weight(b)
                pltpu.sync_copy(buf.at[b], tab.at[jv.at[b]], add=True)

        @pl.loop(NCHF, NCH)
        def _(c):
            off = pl.multiple_of(base + c * CH, 8)
            pltpu.sync_copy(jr.at[pl.ds(off, CH)], jv.at[0])
            pltpu.sync_copy(ir.at[pl.ds(off, CH)], iv.at[0])
            pltpu.sync_copy(er.at[pl.ds(off, CH)], ebuf.at[0])
            pltpu.sync_copy(tn.at[iv.at[0]], buf.at[0])
            weight(0)
            pltpu.sync_copy(buf.at[0], tab.at[jv.at[0]], add=True)

        plsc.subcore_barrier()

        @pl.loop(0, NPT // ZR)
        def _(b):
            r0 = pl.multiple_of(sid * NPT + b * ZR, 8)
            pltpu.sync_copy(tab.at[pl.ds(r0, ZR)], out.at[cid].at[pl.ds(r0, ZR)])

    return k(tab_nodes, j_idx, i_idx, e16)


def _sc_scatter_add(j_idx, val):
    d = val.shape[1]

    @functools.partial(
        pl.kernel,
        mesh=_mesh,
        out_type=jax.ShapeDtypeStruct((NC, N_TAB, d), f32),
        scratch_types=[
            pltpu.VMEM((_NB, CH), i32),
            pltpu.VMEM((_NB, CH, d), f32),
            pltpu.VMEM((ZR, d), f32),
            pltpu.VMEM_SHARED((N_TAB, d), f32),
            pltpu.SemaphoreType.DMA,
            pltpu.SemaphoreType.DMA,
        ],
        compiler_params=_sc_params,
    )
    def k(jr, vr, out, jv, buf, z, tab, ss0, ss1):
        ss = (ss0, ss1)
        cid = lax.axis_index("c")
        sid = lax.axis_index("s")
        wid = sid * NC + cid
        base = wid * EW

        @pl.loop(0, ZR)
        def _(r):
            for c0 in range(d // 16):
                z[r, pl.ds(c0 * 16, 16)] = jnp.zeros((16,), f32)

        @pl.loop(0, NPT // ZR)
        def _(b):
            r0 = pl.multiple_of(sid * NPT + b * ZR, 8)
            pltpu.sync_copy(z, tab.at[pl.ds(r0, ZR)])

        plsc.subcore_barrier()

        @pl.loop(0, NCH - 1, step=2)
        def _(c0):
            offs = [pl.multiple_of(base + (c0 + b) * CH, 8) for b in (0, 1)]
            for b in (0, 1):
                pltpu.async_copy(jr.at[pl.ds(offs[b], CH)], jv.at[b], ss[b])
                pltpu.async_copy(vr.at[pl.ds(offs[b], CH)], buf.at[b], ss[b])
            for b in (0, 1):
                pltpu.make_async_copy(jr.at[pl.ds(offs[b], CH)], jv.at[b], ss[b]).wait()
                pltpu.make_async_copy(vr.at[pl.ds(offs[b], CH)], buf.at[b], ss[b]).wait()
                pltpu.sync_copy(buf.at[b], tab.at[jv.at[b]], add=True)

        off = pl.multiple_of(base + (NCH - 1) * CH, 8)
        pltpu.sync_copy(jr.at[pl.ds(off, CH)], jv.at[0])
        pltpu.sync_copy(vr.at[pl.ds(off, CH)], buf.at[0])
        pltpu.sync_copy(buf.at[0], tab.at[jv.at[0]], add=True)

        plsc.subcore_barrier()

        @pl.loop(0, NPT // ZR)
        def _(b):
            r0 = pl.multiple_of(sid * NPT + b * ZR, 8)
            pltpu.sync_copy(tab.at[pl.ds(r0, ZR)], out.at[cid].at[pl.ds(r0, ZR)])

    return k(j_idx, val)


_BN = 1000
_BE = 3200


def _tc_proj(h, v_t, Wq, bq, Wk, bk, Wvh, bvh, W_Vv):
    nb = N // _BN

    def body(h_r, vt_r, wq, bq_, wk, bk_, wvh, bvh_, wvv,
             q_o, k_o, vh_o, vx_o, vy_o, vz_o):
        hb = h_r[...]
        q_o[...] = jnp.dot(hb, wq[...], preferred_element_type=f32) + bq_[...]
        k_o[...] = jnp.dot(hb, wk[...], preferred_element_type=f32) + bk_[...]
        vh_o[...] = jnp.dot(hb, wvh[...], preferred_element_type=f32) + bvh_[...]
        for c, o in enumerate((vx_o, vy_o, vz_o)):
            o[...] = jnp.dot(vt_r[c], wvv[...], preferred_element_type=f32)

    w_spec = pl.BlockSpec((DIM, DIM), lambda b: (0, 0))
    b_spec = pl.BlockSpec((1, DIM), lambda b: (0, 0))
    return pl.pallas_call(
        body,
        grid=(nb,),
        in_specs=[
            pl.BlockSpec((_BN, DIM), lambda b: (b, 0)),
            pl.BlockSpec((3, _BN, DIM), lambda b: (0, b, 0)),
            w_spec, b_spec, w_spec, b_spec, w_spec, b_spec, w_spec,
        ],
        out_specs=[pl.BlockSpec((_BN, DIM), lambda b: (b, 0))] * 6,
        out_shape=[jax.ShapeDtypeStruct((N, DIM), f32)] * 6,
    )(h, v_t, Wq, bq.reshape(1, DIM), Wk, bk.reshape(1, DIM),
      Wvh, bvh.reshape(1, DIM), W_Vv)


def _tc_edge_mlp(edge_attr, edge_len, w1, b1, w2, b2):
    nb = E // _BE
    ed = edge_attr.shape[1]
    eaT = edge_attr.T
    elT = edge_len.reshape(1, E)

    def body(ea_r, el_r, w1_r, b1_r, w2_r, b2_r, sb_o):
        t = jnp.dot(w1_r[...], ea_r[...], preferred_element_type=f32) + b1_r[...]
        t = t * jax.nn.sigmoid(t)
        sb_o[...] = (jnp.dot(w2_r[...], t, preferred_element_type=f32)
                     + b2_r[...] - el_r[...])

    return pl.pallas_call(
        body,
        grid=(nb,),
        in_specs=[
            pl.BlockSpec((ed, _BE), lambda b: (0, b)),
            pl.BlockSpec((1, _BE), lambda b: (0, b)),
            pl.BlockSpec((ed, ed), lambda b: (0, 0)),
            pl.BlockSpec((ed, 1), lambda b: (0, 0)),
            pl.BlockSpec((H, ed), lambda b: (0, 0)),
            pl.BlockSpec((H, 1), lambda b: (0, 0)),
        ],
        out_specs=pl.BlockSpec((H, _BE), lambda b: (0, b)),
        out_shape=jax.ShapeDtypeStruct((H, E), f32),
    )(eaT, elT, w1.T, b1.reshape(ed, 1), w2.T, b2.reshape(1, H).T)


def _tc_scores(qj, ki, sb):
    nb = E // _BE
    scale = 1.0 / float(DK) ** 0.5

    def body(qj_r, ki_r, sb_r, sc_o, m_o, macc):
        b = pl.program_id(0)

        @pl.when(b == 0)
        def _():
            macc[0, 0] = -jnp.inf

        prod = qj_r[...] * ki_r[...]
        rr = lax.broadcasted_iota(i32, (DIM, H), 0) // DK
        cc = lax.broadcasted_iota(i32, (DIM, H), 1)
        hsum = jnp.where(rr == cc, scale, 0.0).astype(f32)
        s3 = (jnp.dot(prod, hsum, preferred_element_type=f32,
                      precision=lax.Precision.HIGHEST) + sb_r[...].T)
        sc_o[...] = s3
        macc[0, 0] = jnp.maximum(macc[0, 0], jnp.max(s3))

        @pl.when(b == nb - 1)
        def _():
            m_o[0, 0] = macc[0, 0]

    return pl.pallas_call(
        body,
        grid=(nb,),
        in_specs=[
            pl.BlockSpec((_BE, DIM), lambda b: (b, 0)),
            pl.BlockSpec((_BE, DIM), lambda b: (b, 0)),
            pl.BlockSpec((H, _BE), lambda b: (0, b)),
        ],
        out_specs=[
            pl.BlockSpec((_BE, H), lambda b: (b, 0)),
            pl.BlockSpec(memory_space=pltpu.SMEM),
        ],
        out_shape=[
            jax.ShapeDtypeStruct((E, H), f32),
            jax.ShapeDtypeStruct((1, 1), f32),
        ],
        scratch_shapes=[pltpu.SMEM((1, 1), f32)],
    )(qj, ki, sb)


def _tc_exp(scores, m):
    nb = E // _BE

    def body(sc_r, m_r, e_o):
        ex = jnp.exp(sc_r[...] - m_r[0, 0])
        e_o[...] = jnp.concatenate([ex, jnp.zeros_like(ex)], axis=1)

    return pl.pallas_call(
        body,
        grid=(nb,),
        in_specs=[
            pl.BlockSpec((_BE, H), lambda b: (b, 0)),
            pl.BlockSpec(memory_space=pltpu.SMEM),
        ],
        out_specs=pl.BlockSpec((_BE, 16), lambda b: (b, 0)),
        out_shape=jax.ShapeDtypeStruct((E, 16), f32),
    )(scores, m)


def _tc_final(s_p, h_p, vx_p, vy_p, vz_p, W_Oh, W_Ov):
    bn = 1024
    nb = N_TAB // bn

    def body(s_r, h_r, vx_r, vy_r, vz_r, woh, wov, dh_o, dv_o):
        s = (s_r[0] + s_r[1])[:, :H] + 1e-16
        rr = lax.broadcasted_iota(i32, (H, DIM), 0)
        cc = lax.broadcasted_iota(i32, (H, DIM), 1) // DK
        bmat = jnp.where(rr == cc, 1.0, 0.0).astype(f32)
        rep = jnp.dot(s, bmat, preferred_element_type=f32,
                      precision=lax.Precision.HIGHEST)
        hagg = (h_r[0] + h_r[1]) / rep
        dh_o[...] = jnp.dot(hagg, woh[...], preferred_element_type=f32)
        for c, v_r in enumerate((vx_r, vy_r, vz_r)):
            vc = (v_r[0] + v_r[1]) / rep
            dv_o[c] = jnp.dot(vc, wov[...], preferred_element_type=f32)

    part_spec = pl.BlockSpec((NC, bn, DIM), lambda b: (0, b, 0))
    return pl.pallas_call(
        body,
        grid=(nb,),
        in_specs=[
            pl.BlockSpec((NC, bn, 16), lambda b: (0, b, 0)),
            part_spec, part_spec, part_spec, part_spec,
            pl.BlockSpec((DIM, DIM), lambda b: (0, 0)),
            pl.BlockSpec((DIM, DIM), lambda b: (0, 0)),
        ],
        out_specs=[
            pl.BlockSpec((bn, DIM), lambda b: (b, 0)),
            pl.BlockSpec((3, bn, DIM), lambda b: (0, b, 0)),
        ],
        out_shape=[
            jax.ShapeDtypeStruct((N_TAB, DIM), f32),
            jax.ShapeDtypeStruct((3, N_TAB, DIM), f32),
        ],
    )(s_p, h_p, vx_p, vy_p, vz_p, W_Oh, W_Ov)


def kernel(h, v, edge_index, edge_attr, edge_len, Wq, bq, Wk, bk, Wvh, bvh,
           W_Vv, W_Oh, W_Ov, mlp_w1, mlp_b1, mlp_w2, mlp_b2):
    i_idx = edge_index[0]
    j_idx = edge_index[1]
    v_t = jnp.transpose(v, (2, 0, 1))

    q_t, k_t, vh_t, vvx_t, vvy_t, vvz_t = _tc_proj(
        h, v_t, Wq, bq, Wk, bk, Wvh, bvh, W_Vv)
    sb = _tc_edge_mlp(edge_attr, edge_len, mlp_w1, mlp_b1, mlp_w2, mlp_b2)

    qj, ki = _sc_gather2(q_t, k_t, j_idx, i_idx)

    scores, m = _tc_scores(qj, ki, sb)
    e16 = _tc_exp(scores, m)

    s_p = _sc_scatter_add(j_idx, e16)
    h_p = _sc_gather_weight_scatter(vh_t, j_idx, i_idx, e16)
    vx_p = _sc_gather_weight_scatter(vvx_t, j_idx, i_idx, e16)
    vy_p = _sc_gather_weight_scatter(vvy_t, j_idx, i_idx, e16)
    vz_p = _sc_gather_weight_scatter(vvz_t, j_idx, i_idx, e16)

    dh_pad, dv3 = _tc_final(s_p, h_p, vx_p, vy_p, vz_p, W_Oh, W_Ov)

    dh = dh_pad[:N]
    dv = jnp.transpose(dv3, (1, 2, 0))[:N]
    return (dh, dv)


# speedup vs baseline: 1.0122x; 1.0122x over previous
import functools

import jax
import jax.numpy as jnp
from jax import lax
from jax.experimental import pallas as pl
from jax.experimental.pallas import tpu as pltpu
from jax.experimental.pallas import tpu_sc as plsc

f32 = jnp.float32
i32 = jnp.int32

N = 10000
E = 320000
DIM = 128
H = 8
DK = 16

NC = 2
NS = 16
NW = NC * NS
EW = E // NW
CH = 80
NCH = EW // CH

N_TAB = 10240
NPT = N_TAB // NS
ZR = 128

_mesh = plsc.VectorSubcoreMesh(core_axis_name="c", subcore_axis_name="s")
_sc_params = pltpu.CompilerParams(use_tc_tiling_on_sc=False)


_NB = 2
_NCH0 = (NCH // _NB) * _NB


def _sc_gather2(tab1, tab2, idx1, idx2):
    d1 = tab1.shape[1]
    d2 = tab2.shape[1]

    @functools.partial(
        pl.kernel,
        mesh=_mesh,
        out_type=[
            jax.ShapeDtypeStruct((E, d1), f32),
            jax.ShapeDtypeStruct((E, d2), f32),
        ],
        scratch_types=[
            pltpu.VMEM((_NB, CH), i32),
            pltpu.VMEM((_NB, CH), i32),
            pltpu.VMEM((_NB, CH, d1), f32),
            pltpu.VMEM((_NB, CH, d2), f32),
            pltpu.SemaphoreType.DMA,
            pltpu.SemaphoreType.DMA,
            pltpu.SemaphoreType.DMA,
            pltpu.SemaphoreType.DMA,
            pltpu.SemaphoreType.DMA,
            pltpu.SemaphoreType.DMA,
        ],
        compiler_params=_sc_params,
    )
    def k(t1, t2, x1, x2, o1, o2, iv1, iv2, b1, b2, sg0, sg1, so0, so1,
          sx0, sx1):
        sg = (sg0, sg1)
        so = (so0, so1)
        sx = (sx0, sx1)
        wid = lax.axis_index("s") * NC + lax.axis_index("c")
        base = wid * EW

        def drain_out(b, off):
            pltpu.make_async_copy(b1.at[b], o1.at[pl.ds(off, CH)], so[b]).wait()
            pltpu.make_async_copy(b2.at[b], o2.at[pl.ds(off, CH)], so[b]).wait()

        def chunk_body(c0, first, width):
            offs = [pl.multiple_of(base + (c0 + b) * CH, 8) for b in range(width)]
            for b in range(width):
                pltpu.async_copy(x1.at[pl.ds(offs[b], CH)], iv1.at[b], sx[b])
                pltpu.async_copy(x2.at[pl.ds(offs[b], CH)], iv2.at[b], sx[b])
            for b in range(width):
                if not first:
                    @pl.when(c0 > 0)
                    def _(b=b, off=offs[b]):
                        drain_out(b, off)
                pltpu.make_async_copy(x1.at[pl.ds(offs[b], CH)], iv1.at[b], sx[b]).wait()
                pltpu.make_async_copy(x2.at[pl.ds(offs[b], CH)], iv2.at[b], sx[b]).wait()
                pltpu.async_copy(t1.at[iv1.at[b]], b1.at[b], sg[b])
                pltpu.async_copy(t2.at[iv2.at[b]], b2.at[b], sg[b])
            for b in range(width):
                pltpu.make_async_copy(t1.at[iv1.at[b]], b1.at[b], sg[b]).wait()
                pltpu.make_async_copy(t2.at[iv2.at[b]], b2.at[b], sg[b]).wait()
                pltpu.async_copy(b1.at[b], o1.at[pl.ds(offs[b], CH)], so[b])
                pltpu.async_copy(b2.at[b], o2.at[pl.ds(offs[b], CH)], so[b])

        @pl.loop(0, _NCH0, step=_NB)
        def _(c0):
            chunk_body(c0, False, _NB)

        for b in range(_NB):
            off = pl.multiple_of(base + (_NCH0 - _NB + b) * CH, 8)
            drain_out(b, off)

        if _NCH0 < NCH:
            chunk_body(_NCH0, True, NCH - _NCH0)
            for b in range(NCH - _NCH0):
                off = pl.multiple_of(base + (_NCH0 + b) * CH, 8)
                drain_out(b, off)

    return k(tab1, tab2, idx1, idx2)


def _sc_gather_weight_scatter(tab_nodes, j_idx, i_idx, e16):
    NBF = 2
    NCHF = (NCH // NBF) * NBF

    @functools.partial(
        pl.kernel,
        mesh=_mesh,
        out_type=jax.ShapeDtypeStruct((NC, N_TAB, DIM), f32),
        scratch_types=[
            pltpu.VMEM((NBF, CH), i32),
            pltpu.VMEM((NBF, CH), i32),
            pltpu.VMEM((NBF, CH, 16), f32),
            pltpu.VMEM((NBF, CH, DIM), f32),
            pltpu.VMEM((ZR, DIM), f32),
            pltpu.VMEM_SHARED((N_TAB, DIM), f32),
        ] + [pltpu.SemaphoreType.DMA] * (2 * NBF),
        compiler_params=_sc_params,
    )
    def k(tn, jr, ir, er, out, jv, iv, ebuf, buf, z, tab, *sems):
        sl = sems[:NBF]
        sg = sems[NBF:2 * NBF]
        cid = lax.axis_index("c")
        sid = lax.axis_index("s")
        wid = sid * NC + cid
        base = wid * EW

        @pl.loop(0, ZR)
        def _(r):
            for c0 in range(DIM // 16):
                z[r, pl.ds(c0 * 16, 16)] = jnp.zeros((16,), f32)

        @pl.loop(0, NPT // ZR)
        def _(b):
            r0 = pl.multiple_of(sid * NPT + b * ZR, 8)
            pltpu.sync_copy(z, tab.at[pl.ds(r0, ZR)])

        plsc.subcore_barrier()

        def weight(b):
            @pl.loop(0, CH)
            def _(r):
                ev = ebuf[b, r]
                for h in range(H):
                    cols = pl.ds(h * DK, DK)
                    buf[b, r, cols] = buf[b, r, cols] * ev[h]

        def fire_loads(b, off):
            pltpu.async_copy(jr.at[pl.ds(off, CH)], jv.at[b], sl[b])
            pltpu.async_copy(ir.at[pl.ds(off, CH)], iv.at[b], sl[b])
            pltpu.async_copy(er.at[pl.ds(off, CH)], ebuf.at[b], sl[b])

        def wait_loads(b, off):
            pltpu.make_async_copy(jr.at[pl.ds(off, CH)], jv.at[b], sl[b]).wait()
            pltpu.make_async_copy(ir.at[pl.ds(off, CH)], iv.at[b], sl[b]).wait()
            pltpu.make_async_copy(er.at[pl.ds(off, CH)], ebuf.at[b], sl[b]).wait()

        @pl.loop(0, NCHF, step=NBF)
        def _(c0):
            offs = [pl.multiple_of(base + (c0 + b) * CH, 8) for b in range(NBF)]
            for b in range(NBF):
                fire_loads(b, offs[b])
            for b in range(NBF):
                wait_loads(b, offs[b])
                pltpu.async_copy(tn.at[iv.at[b]], buf.at[b], sg[b])
            for b in range(NBF):
                pltpu.make_async_copy(tn.at[iv.at[b]], buf.at[b], sg[b]).wait()
                weight(b)
                pltpu.sync_copy(buf.at[b], tab.at[jv.at[b]], add=True)

        @pl.loop(NCHF, NCH)
        def _(c):
            off = pl.multiple_of(base + c * CH, 8)
            pltpu.sync_copy(jr.at[pl.ds(off, CH)], jv.at[0])
            pltpu.sync_copy(ir.at[pl.ds(off, CH)], iv.at[0])
            pltpu.sync_copy(er.at[pl.ds(off, CH)], ebuf.at[0])
            pltpu.sync_copy(tn.at[iv.at[0]], buf.at[0])
            weight(0)
            pltpu.sync_copy(buf.at[0], tab.at[jv.at[0]], add=True)

        plsc.subcore_barrier()

        @pl.loop(0, NPT // ZR)
        def _(b):
            r0 = pl.multiple_of(sid * NPT + b * ZR, 8)
            pltpu.sync_copy(tab.at[pl.ds(r0, ZR)], out.at[cid].at[pl.ds(r0, ZR)])

    return k(tab_nodes, j_idx, i_idx, e16)


def _sc_scatter_add(j_idx, val):
    d = val.shape[1]

    @functools.partial(
        pl.kernel,
        mesh=_mesh,
        out_type=jax.ShapeDtypeStruct((NC, N_TAB, d), f32),
        scratch_types=[
            pltpu.VMEM((_NB, CH), i32),
            pltpu.VMEM((_NB, CH, d), f32),
            pltpu.VMEM((ZR, d), f32),
            pltpu.VMEM_SHARED((N_TAB, d), f32),
            pltpu.SemaphoreType.DMA,
            pltpu.SemaphoreType.DMA,
        ],
        compiler_params=_sc_params,
    )
    def k(jr, vr, out, jv, buf, z, tab, ss0, ss1):
        ss = (ss0, ss1)
        cid = lax.axis_index("c")
        sid = lax.axis_index("s")
        wid = sid * NC + cid
        base = wid * EW

        @pl.loop(0, ZR)
        def _(r):
            for c0 in range(d // 16):
                z[r, pl.ds(c0 * 16, 16)] = jnp.zeros((16,), f32)

        @pl.loop(0, NPT // ZR)
        def _(b):
            r0 = pl.multiple_of(sid * NPT + b * ZR, 8)
            pltpu.sync_copy(z, tab.at[pl.ds(r0, ZR)])

        plsc.subcore_barrier()

        @pl.loop(0, NCH - 1, step=2)
        def _(c0):
            offs = [pl.multiple_of(base + (c0 + b) * CH, 8) for b in (0, 1)]
            for b in (0, 1):
                pltpu.async_copy(jr.at[pl.ds(offs[b], CH)], jv.at[b], ss[b])
                pltpu.async_copy(vr.at[pl.ds(offs[b], CH)], buf.at[b], ss[b])
            for b in (0, 1):
                pltpu.make_async_copy(jr.at[pl.ds(offs[b], CH)], jv.at[b], ss[b]).wait()
                pltpu.make_async_copy(vr.at[pl.ds(offs[b], CH)], buf.at[b], ss[b]).wait()
                pltpu.sync_copy(buf.at[b], tab.at[jv.at[b]], add=True)

        off = pl.multiple_of(base + (NCH - 1) * CH, 8)
        pltpu.sync_copy(jr.at[pl.ds(off, CH)], jv.at[0])
        pltpu.sync_copy(vr.at[pl.ds(off, CH)], buf.at[0])
        pltpu.sync_copy(buf.at[0], tab.at[jv.at[0]], add=True)

        plsc.subcore_barrier()

        @pl.loop(0, NPT // ZR)
        def _(b):
            r0 = pl.multiple_of(sid * NPT + b * ZR, 8)
            pltpu.sync_copy(tab.at[pl.ds(r0, ZR)], out.at[cid].at[pl.ds(r0, ZR)])

    return k(j_idx, val)


_BN = 1000
_BE = 3200


def _tc_proj(h, v_t, Wq, bq, Wk, bk, Wvh, bvh, W_Vv):
    nb = N // _BN

    def body(h_r, vt_r, wq, bq_, wk, bk_, wvh, bvh_, wvv,
             q_o, k_o, vh_o, vx_o, vy_o, vz_o):
        hb = h_r[...]
        q_o[...] = jnp.dot(hb, wq[...], preferred_element_type=f32) + bq_[...]
        k_o[...] = jnp.dot(hb, wk[...], preferred_element_type=f32) + bk_[...]
        vh_o[...] = jnp.dot(hb, wvh[...], preferred_element_type=f32) + bvh_[...]
        for c, o in enumerate((vx_o, vy_o, vz_o)):
            o[...] = jnp.dot(vt_r[c], wvv[...], preferred_element_type=f32)

    w_spec = pl.BlockSpec((DIM, DIM), lambda b: (0, 0))
    b_spec = pl.BlockSpec((1, DIM), lambda b: (0, 0))
    return pl.pallas_call(
        body,
        grid=(nb,),
        in_specs=[
            pl.BlockSpec((_BN, DIM), lambda b: (b, 0)),
            pl.BlockSpec((3, _BN, DIM), lambda b: (0, b, 0)),
            w_spec, b_spec, w_spec, b_spec, w_spec, b_spec, w_spec,
        ],
        out_specs=[pl.BlockSpec((_BN, DIM), lambda b: (b, 0))] * 6,
        out_shape=[jax.ShapeDtypeStruct((N, DIM), f32)] * 6,
    )(h, v_t, Wq, bq.reshape(1, DIM), Wk, bk.reshape(1, DIM),
      Wvh, bvh.reshape(1, DIM), W_Vv)


def _tc_edge_mlp(edge_attr, edge_len, w1, b1, w2, b2):
    nb = E // _BE
    ed = edge_attr.shape[1]
    eaT = edge_attr.T
    elT = edge_len.reshape(1, E)

    def body(ea_r, el_r, w1_r, b1_r, w2_r, b2_r, sb_o):
        t = jnp.dot(w1_r[...], ea_r[...], preferred_element_type=f32) + b1_r[...]
        t = t * jax.nn.sigmoid(t)
        sb_o[...] = (jnp.dot(w2_r[...], t, preferred_element_type=f32)
                     + b2_r[...] - el_r[...])

    return pl.pallas_call(
        body,
        grid=(nb,),
        in_specs=[
            pl.BlockSpec((ed, _BE), lambda b: (0, b)),
            pl.BlockSpec((1, _BE), lambda b: (0, b)),
            pl.BlockSpec((ed, ed), lambda b: (0, 0)),
            pl.BlockSpec((ed, 1), lambda b: (0, 0)),
            pl.BlockSpec((H, ed), lambda b: (0, 0)),
            pl.BlockSpec((H, 1), lambda b: (0, 0)),
        ],
        out_specs=pl.BlockSpec((H, _BE), lambda b: (0, b)),
        out_shape=jax.ShapeDtypeStruct((H, E), f32),
    )(eaT, elT, w1.T, b1.reshape(ed, 1), w2.T, b2.reshape(1, H).T)


def _tc_scores(qj, ki, sb):
    nb = E // _BE
    scale = 1.0 / float(DK) ** 0.5

    def body(qj_r, ki_r, sb_r, sc_o, m_o, macc):
        b = pl.program_id(0)

        @pl.when(b == 0)
        def _():
            macc[0, 0] = -jnp.inf

        prod = qj_r[...] * ki_r[...]
        rr = lax.broadcasted_iota(i32, (DIM, H), 0) // DK
        cc = lax.broadcasted_iota(i32, (DIM, H), 1)
        hsum = jnp.where(rr == cc, scale, 0.0).astype(f32)
        s3 = (jnp.dot(prod, hsum, preferred_element_type=f32,
                      precision=lax.Precision.HIGHEST) + sb_r[...].T)
        sc_o[...] = s3
        macc[0, 0] = jnp.maximum(macc[0, 0], jnp.max(s3))

        @pl.when(b == nb - 1)
        def _():
            m_o[0, 0] = macc[0, 0]

    return pl.pallas_call(
        body,
        grid=(nb,),
        in_specs=[
            pl.BlockSpec((_BE, DIM), lambda b: (b, 0)),
            pl.BlockSpec((_BE, DIM), lambda b: (b, 0)),
            pl.BlockSpec((H, _BE), lambda b: (0, b)),
        ],
        out_specs=[
            pl.BlockSpec((_BE, H), lambda b: (b, 0)),
            pl.BlockSpec(memory_space=pltpu.SMEM),
        ],
        out_shape=[
            jax.ShapeDtypeStruct((E, H), f32),
            jax.ShapeDtypeStruct((1, 1), f32),
        ],
        scratch_shapes=[pltpu.SMEM((1, 1), f32)],
    )(qj, ki, sb)


def _tc_exp(scores, m):
    nb = E // _BE

    def body(sc_r, m_r, e_o):
        ex = jnp.exp(sc_r[...] - m_r[0, 0])
        e_o[...] = jnp.concatenate([ex, jnp.zeros_like(ex)], axis=1)

    return pl.pallas_call(
        body,
        grid=(nb,),
        in_specs=[
            pl.BlockSpec((_BE, H), lambda b: (b, 0)),
            pl.BlockSpec(memory_space=pltpu.SMEM),
        ],
        out_specs=pl.BlockSpec((_BE, 16), lambda b: (b, 0)),
        out_shape=jax.ShapeDtypeStruct((E, 16), f32),
    )(scores, m)


def _tc_final(s_p, h_p, vx_p, vy_p, vz_p, W_Oh, W_Ov):
    bn = 1024
    nb = N_TAB // bn

    def body(s_r, h_r, vx_r, vy_r, vz_r, woh, wov, dh_o, dv_o):
        s = (s_r[0] + s_r[1])[:, :H] + 1e-16
        rr = lax.broadcasted_iota(i32, (H, DIM), 0)
        cc = lax.broadcasted_iota(i32, (H, DIM), 1) // DK
        bmat = jnp.where(rr == cc, 1.0, 0.0).astype(f32)
        rep = jnp.dot(s, bmat, preferred_element_type=f32,
                      precision=lax.Precision.HIGHEST)
        hagg = (h_r[0] + h_r[1]) / rep
        dh_o[...] = jnp.dot(hagg, woh[...], preferred_element_type=f32)
        for c, v_r in enumerate((vx_r, vy_r, vz_r)):
            vc = (v_r[0] + v_r[1]) / rep
            dv_o[c] = jnp.dot(vc, wov[...], preferred_element_type=f32)

    part_spec = pl.BlockSpec((NC, bn, DIM), lambda b: (0, b, 0))
    return pl.pallas_call(
        body,
        grid=(nb,),
        in_specs=[
            pl.BlockSpec((NC, bn, 16), lambda b: (0, b, 0)),
            part_spec, part_spec, part_spec, part_spec,
            pl.BlockSpec((DIM, DIM), lambda b: (0, 0)),
            pl.BlockSpec((DIM, DIM), lambda b: (0, 0)),
        ],
        out_specs=[
            pl.BlockSpec((bn, DIM), lambda b: (b, 0)),
            pl.BlockSpec((3, bn, DIM), lambda b: (0, b, 0)),
        ],
        out_shape=[
            jax.ShapeDtypeStruct((N_TAB, DIM), f32),
            jax.ShapeDtypeStruct((3, N_TAB, DIM), f32),
        ],
    )(s_p, h_p, vx_p, vy_p, vz_p, W_Oh, W_Ov)


def kernel(h, v, edge_index, edge_attr, edge_len, Wq, bq, Wk, bk, Wvh, bvh,
           W_Vv, W_Oh, W_Ov, mlp_w1, mlp_b1, mlp_w2, mlp_b2):
    i_idx = edge_index[0]
    j_idx = edge_index[1]
    v_t = jnp.transpose(v, (2, 0, 1))

    q_t, k_t, vh_t, vvx_t, vvy_t, vvz_t = _tc_proj(
        h, v_t, Wq, bq, Wk, bk, Wvh, bvh, W_Vv)
    sb = _tc_edge_mlp(edge_attr, edge_len, mlp_w1, mlp_b1, mlp_w2, mlp_b2)

    qj, ki = _sc_gather2(q_t, k_t, j_idx, i_idx)

    scores, m = _tc_scores(qj, ki, sb)
    e16 = _tc_exp(scores, m)

    s_p = _sc_scatter_add(j_idx, e16)
    h_p = _sc_gather_weight_scatter(vh_t, j_idx, i_idx, e16)
    vx_p = _sc_gather_weight_scatter(vvx_t, j_idx, i_idx, e16)
    vy_p = _sc_gather_weight_scatter(vvy_t, j_idx, i_idx, e16)
    vz_p = _sc_gather_weight_scatter(vvz_t, j_idx, i_idx, e16)

    dh_pad, dv3 = _tc_final(s_p, h_p, vx_p, vy_p, vz_p, W_Oh, W_Ov)

    dh = dh_pad[:N]
    dv = jnp.transpose(dv3, (1, 2, 0))[:N]
    return (dh, dv)
